# baseline (speedup 1.0000x reference)
import jax
import jax.numpy as jnp
from jax.experimental import pallas as pl
from jax.experimental.pallas import tpu as pltpu

_T, _P, _O, _C, _F, _K = 16, 64, 4, 32, 64, 128


_PB = 32


def _embed_body(x_ref, w_ref, b_ref, out_ref):
    for o in range(_O):
        xt = x_ref[0, :, o, :, :].transpose(1, 0, 2).reshape(_C * _PB, _F)
        acc = jax.lax.dot_general(
            xt, w_ref[o], (((1,), (0,)), ((), ())),
            preferred_element_type=jnp.float32)
        acc = acc + b_ref[o][None, :]
        acc = jnp.where(acc >= 0, acc, 0.1 * acc)
        out_ref[0, o * _C:(o + 1) * _C] = acc.reshape(_C, _PB, _K)


def kernel(x, W, b):
    outs = pl.pallas_call(
        _embed_body,
        grid=(_T, _P // _PB),
        in_specs=[
            pl.BlockSpec((1, _PB, _O, _C, _F), lambda t, p: (t, p, 0, 0, 0)),
            pl.BlockSpec((_O, _F, _K), lambda t, p: (0, 0, 0)),
            pl.BlockSpec((_O, _K), lambda t, p: (0, 0)),
        ],
        out_specs=pl.BlockSpec((1, _O * _C, _PB, _K),
                               lambda t, p: (t, 0, p, 0)),
        out_shape=jax.ShapeDtypeStruct((_T, _O * _C, _P, _K), jnp.float32),
        compiler_params=pltpu.CompilerParams(
            dimension_semantics=("parallel", "parallel")),
    )(x, W, b)
    objCounts = jnp.full((_T, _P), _O * _C, dtype=jnp.int32)
    return outs, objCounts


# speedup vs baseline: 1.1930x; 1.1930x over previous
import jax
import jax.numpy as jnp
from jax.experimental import pallas as pl
from jax.experimental.pallas import tpu as pltpu

_T, _P, _O, _C, _F, _K = 16, 64, 4, 32, 64, 128


_TB = 2


def _embed_body(x_ref, w_ref, b_ref, out_ref):
    for t in range(_TB):
        for o in range(_O):
            xt = x_ref[t, :, o, :, :].transpose(1, 0, 2).reshape(_C * _P, _F)
            acc = jax.lax.dot_general(
                xt, w_ref[o], (((1,), (0,)), ((), ())),
                preferred_element_type=jnp.float32)
            acc = acc + b_ref[o][None, :]
            acc = jnp.where(acc >= 0, acc, 0.1 * acc)
            out_ref[t, o * _C:(o + 1) * _C] = acc.reshape(_C, _P, _K)


def kernel(x, W, b):
    outs = pl.pallas_call(
        _embed_body,
        grid=(_T // _TB,),
        in_specs=[
            pl.BlockSpec((_TB, _P, _O, _C, _F), lambda t: (t, 0, 0, 0, 0)),
            pl.BlockSpec((_O, _F, _K), lambda t: (0, 0, 0)),
            pl.BlockSpec((_O, _K), lambda t: (0, 0)),
        ],
        out_specs=pl.BlockSpec((_TB, _O * _C, _P, _K), lambda t: (t, 0, 0, 0)),
        out_shape=jax.ShapeDtypeStruct((_T, _O * _C, _P, _K), jnp.float32),
        compiler_params=pltpu.CompilerParams(
            dimension_semantics=("parallel",)),
    )(x, W, b)
    objCounts = jnp.full((_T, _P), _O * _C, dtype=jnp.int32)
    return outs, objCounts
